# baseline (speedup 1.0000x reference)
import jax
import jax.numpy as jnp
from jax import lax
from jax.experimental import pallas as pl
from jax.experimental.pallas import tpu as pltpu

_B = 256
_NH = 40
_NV = 40
_N = _NH * _NV
_THR = 0.025
_CB = 32
_NC = _B // _CB


def _loss_kernel(p0_ref, lab_ref, out_ref, pacc, nacc, mask, npos):
    c = pl.program_id(0)

    @pl.when(c == 0)
    def _():
        pacc[...] = jnp.zeros((1, _N), jnp.float32)
        nacc[...] = jnp.zeros((1, _N), jnp.float32)

    p0 = p0_ref[...]
    t = jnp.exp(1.0 - 2.0 * p0)
    conf = 1.0 / (1.0 + t)
    pos_log = -jnp.log(conf + 1e-8)
    neg_log = -jnp.log(1.0 - conf + 1e-8)
    pacc[...] += jnp.sum(pos_log, axis=0, keepdims=True)
    nacc[...] += jnp.sum(neg_log, axis=0, keepdims=True)

    idx = lax.broadcasted_iota(jnp.int32, (1, _N), 1)
    gx = (idx // _NV).astype(jnp.float32) * (1.0 / _NH) + (0.5 / _NH)
    gy = (idx % _NV).astype(jnp.float32) * (1.0 / _NV) + (0.5 / _NV)
    lx = lab_ref[:, 0:1]
    ly = lab_ref[:, 1:2]
    dist = jnp.abs(gx - lx) + jnp.abs(gy - ly)
    pos = (dist <= _THR).astype(jnp.float32)
    mask[pl.ds(c * _CB, _CB), :] = pos
    npos[pl.ds(c * _CB, _CB), :] = jnp.sum(pos, axis=1, keepdims=True)

    @pl.when(c == _NC - 1)
    def _():
        P = pacc[...]
        Ng = nacc[...]
        T = jnp.sum(Ng)
        m = mask[...]
        num_pos = npos[...]
        num_neg = jnp.float32(_N) - num_pos
        s_pos = jnp.sum(P * m, axis=1, keepdims=True)
        s_negpos = jnp.sum(Ng * m, axis=1, keepdims=True)
        loss = s_pos / num_pos + 3.0 * (T - s_negpos) / num_neg
        out_ref[0, 0] = jnp.sum(loss) * (1.0 / _B)


def kernel(predictions, labels, device):
    p0 = predictions[:, 0, :]
    out = pl.pallas_call(
        _loss_kernel,
        grid=(_NC,),
        in_specs=[
            pl.BlockSpec((_CB, _N), lambda c: (c, 0)),
            pl.BlockSpec((_CB, 2), lambda c: (c, 0)),
        ],
        out_specs=pl.BlockSpec(memory_space=pltpu.SMEM),
        out_shape=jax.ShapeDtypeStruct((1, 1), jnp.float32),
        scratch_shapes=[
            pltpu.VMEM((1, _N), jnp.float32),
            pltpu.VMEM((1, _N), jnp.float32),
            pltpu.VMEM((_B, _N), jnp.float32),
            pltpu.VMEM((_B, 1), jnp.float32),
        ],
    )(p0, labels)
    return out[0, 0]


# speedup vs baseline: 1.0012x; 1.0012x over previous
import jax
import jax.numpy as jnp
from jax import lax
from jax.experimental import pallas as pl
from jax.experimental.pallas import tpu as pltpu

_B = 256
_NH = 40
_NV = 40
_N = _NH * _NV
_THR = 0.025
_CB = 32
_NC = _B // _CB


def _loss_kernel(p0_ref, lab_ref, out_ref, pacc, nacc):
    c = pl.program_id(0)

    @pl.when(c == 0)
    def _():
        pacc[...] = jnp.zeros((1, _N), jnp.float32)
        nacc[...] = jnp.zeros((1, _N), jnp.float32)

    p0 = p0_ref[...]
    t = jnp.exp(1.0 - 2.0 * p0)
    conf = 1.0 / (1.0 + t)
    pos_log = -jnp.log(conf + 1e-8)
    neg_log = -jnp.log(1.0 - conf + 1e-8)
    pacc[...] += jnp.sum(pos_log, axis=0, keepdims=True)
    nacc[...] += jnp.sum(neg_log, axis=0, keepdims=True)

    @pl.when(c == _NC - 1)
    def _():
        P = pacc[...]
        Ng = nacc[...]
        T = jnp.sum(Ng)

        idx = lax.broadcasted_iota(jnp.int32, (1, _N), 1)
        gx = (idx // _NV).astype(jnp.float32) * (1.0 / _NH) + (0.5 / _NH)
        gy = (idx % _NV).astype(jnp.float32) * (1.0 / _NV) + (0.5 / _NV)

        lx = lab_ref[:, 0:1]
        ly = lab_ref[:, 1:2]
        dist = jnp.abs(gx - lx) + jnp.abs(gy - ly)
        pos = (dist <= _THR).astype(jnp.float32)

        num_pos = jnp.sum(pos, axis=1, keepdims=True)
        num_neg = jnp.float32(_N) - num_pos
        s_pos = jnp.sum(P * pos, axis=1, keepdims=True)
        s_negpos = jnp.sum(Ng * pos, axis=1, keepdims=True)

        loss = s_pos / num_pos + 3.0 * (T - s_negpos) / num_neg
        out_ref[0, 0] = jnp.sum(loss) * (1.0 / _B)


def kernel(predictions, labels, device):
    p0 = predictions[:, 0, :]
    out = pl.pallas_call(
        _loss_kernel,
        grid=(_NC,),
        in_specs=[
            pl.BlockSpec((_CB, _N), lambda c: (c, 0)),
            pl.BlockSpec((_B, 2), lambda c: (0, 0)),
        ],
        out_specs=pl.BlockSpec(memory_space=pltpu.SMEM),
        out_shape=jax.ShapeDtypeStruct((1, 1), jnp.float32),
        scratch_shapes=[
            pltpu.VMEM((1, _N), jnp.float32),
            pltpu.VMEM((1, _N), jnp.float32),
        ],
    )(p0, labels)
    return out[0, 0]


# speedup vs baseline: 1.2884x; 1.2868x over previous
import jax
import jax.numpy as jnp
from jax import lax
from jax.experimental import pallas as pl
from jax.experimental.pallas import tpu as pltpu

_B = 256
_NH = 40
_NV = 40
_N = _NH * _NV
_THR = 0.025
_CB = 128
_NC = _B // _CB


def _loss_kernel(p0_ref, lab_ref, out_ref, pacc, nacc):
    c = pl.program_id(0)

    @pl.when(c == 0)
    def _():
        pacc[...] = jnp.zeros((1, _N), jnp.float32)
        nacc[...] = jnp.zeros((1, _N), jnp.float32)

    p0 = p0_ref[...]
    t = jnp.exp(1.0 - 2.0 * p0)
    conf = 1.0 / (1.0 + t)
    pos_log = -jnp.log(conf + 1e-8)
    neg_log = -jnp.log(1.0 - conf + 1e-8)
    pacc[...] += jnp.sum(pos_log, axis=0, keepdims=True)
    nacc[...] += jnp.sum(neg_log, axis=0, keepdims=True)

    @pl.when(c == _NC - 1)
    def _():
        P = pacc[...]
        Ng = nacc[...]
        T = jnp.sum(Ng)

        idx = lax.broadcasted_iota(jnp.int32, (1, _N), 1)
        gx = (idx // _NV).astype(jnp.float32) * (1.0 / _NH) + (0.5 / _NH)
        gy = (idx % _NV).astype(jnp.float32) * (1.0 / _NV) + (0.5 / _NV)

        lx = lab_ref[:, 0:1]
        ly = lab_ref[:, 1:2]
        dist = jnp.abs(gx - lx) + jnp.abs(gy - ly)
        pos = (dist <= _THR).astype(jnp.float32)

        num_pos = jnp.sum(pos, axis=1, keepdims=True)
        num_neg = jnp.float32(_N) - num_pos
        s_pos = jnp.sum(P * pos, axis=1, keepdims=True)
        s_negpos = jnp.sum(Ng * pos, axis=1, keepdims=True)

        loss = s_pos / num_pos + 3.0 * (T - s_negpos) / num_neg
        out_ref[0, 0] = jnp.sum(loss) * (1.0 / _B)


def kernel(predictions, labels, device):
    p0 = predictions[:, 0, :]
    out = pl.pallas_call(
        _loss_kernel,
        grid=(_NC,),
        in_specs=[
            pl.BlockSpec((_CB, _N), lambda c: (c, 0)),
            pl.BlockSpec((_B, 2), lambda c: (0, 0)),
        ],
        out_specs=pl.BlockSpec(memory_space=pltpu.SMEM),
        out_shape=jax.ShapeDtypeStruct((1, 1), jnp.float32),
        scratch_shapes=[
            pltpu.VMEM((1, _N), jnp.float32),
            pltpu.VMEM((1, _N), jnp.float32),
        ],
    )(p0, labels)
    return out[0, 0]
